# baseline (speedup 1.0000x reference)
import functools

import jax
import jax.numpy as jnp
from jax import lax
from jax.experimental import pallas as pl
from jax.experimental.pallas import tpu as pltpu
from jax.experimental.pallas import tpu_sc as plsc

_NC = 2
_NS = 16
_NW = _NC * _NS
_LANES = 16
_CHUNK = 4096
_NBUF = 2


@functools.lru_cache(maxsize=None)
def _make_spmm(F, N, E2eff, mode="pair", chunk=_CHUNK):
    fpw = -(-F // _NW)
    nwact = -(-F // fpw)
    nchunks = E2eff // chunk
    groups = chunk // _LANES

    mesh = plsc.VectorSubcoreMesh(
        core_axis_name="c", subcore_axis_name="s",
        num_cores=_NC, num_subcores=_NS)

    def make_helpers(rc_hbm, vals_hbm, rc_v, v_v, sems):
        def start(ch, b):
            off = ch * chunk
            pltpu.async_copy(rc_hbm.at[pl.ds(off, chunk)], rc_v.at[b], sems[b])
            pltpu.async_copy(vals_hbm.at[pl.ds(off, chunk)], v_v.at[b], sems[b])

        def drain(b):
            pltpu.make_async_copy(rc_hbm.at[pl.ds(0, chunk)], rc_v.at[b], sems[b]).wait()
            pltpu.make_async_copy(vals_hbm.at[pl.ds(0, chunk)], v_v.at[b], sems[b]).wait()

        def zero(dst_v):
            z = jnp.zeros((_LANES,), jnp.float32)

            @plsc.parallel_loop(0, N // _LANES, unroll=8)
            def zbody(i):
                for j in range(fpw):
                    dst_v[j, pl.ds(i * _LANES, _LANES)] = z

        def edge_pass(src_v, dst_v, ch_base=0, nch=nchunks):
            start(ch_base, 0)

            def compute(b):
                @plsc.parallel_loop(0, groups, unroll=8)
                def body(gi):
                    base = gi * _LANES
                    rc = rc_v[b, pl.ds(base, _LANES)]
                    rr = lax.shift_right_logical(rc, 14)
                    cc = lax.bitwise_and(rc, 16383)
                    vv = v_v[b, pl.ds(base, _LANES)]
                    gs = []
                    for j in range(fpw):
                        jf = jnp.full((_LANES,), j, jnp.int32)
                        gs.append(plsc.load_gather(src_v, [jf, cc]) * vv)
                    for j in range(fpw):
                        jf = jnp.full((_LANES,), j, jnp.int32)
                        plsc.addupdate_scatter(dst_v, [jf, rr], gs[j])

            def outer(g, carry):
                for b in range(_NBUF):
                    ch = g * _NBUF + b

                    @pl.when(ch + 1 < nch)
                    def _():
                        start(ch_base + ch + 1, 1 - b)

                    drain(b)
                    compute(b)
                return carry
            lax.fori_loop(0, nch // _NBUF, outer, 0)

        return zero, edge_pass

    if mode == "pair_split":
        fsc = F // _NC
        half = nchunks // 2

        @functools.partial(
            pl.kernel,
            out_type=(jax.ShapeDtypeStruct((F, N), jnp.float32),
                      jax.ShapeDtypeStruct((F, N), jnp.float32)),
            mesh=mesh,
            compiler_params=pltpu.CompilerParams(needs_layout_passes=False),
            scratch_types=[
                pltpu.VMEM((1, N), jnp.float32),
                pltpu.VMEM((1, N), jnp.float32),
                pltpu.VMEM((1, N), jnp.float32),
                pltpu.VMEM_SHARED((_NS, N), jnp.float32),
                pltpu.VMEM((_NBUF, chunk), jnp.int32),
                pltpu.VMEM((_NBUF, chunk), jnp.float32),
                pltpu.SemaphoreType.DMA,
                pltpu.SemaphoreType.DMA,
            ],
        )
        def spmm(x_hbm, rc_hbm, vals_hbm, y1_hbm, y2_hbm,
                 a_v, b_v, t_v, sh, rc_v, v_v, sem0, sem1):
            sid = lax.axis_index("s")
            cid = lax.axis_index("c")
            zero, edge_pass = make_helpers(rc_hbm, vals_hbm, rc_v, v_v,
                                           (sem0, sem1))
            owner = sid < fsc
            helper = jnp.logical_and(sid >= 8, sid < 8 + fsc)
            active = jnp.logical_or(owner, helper)
            floc = jnp.where(owner, sid, sid - 8)
            f = cid * fsc + floc
            ch0 = jnp.where(owner, 0, half)

            def merge(dst_v):
                @pl.when(active)
                def _():
                    pltpu.sync_copy(dst_v, sh.at[pl.ds(sid, 1)])
                plsc.subcore_barrier()

                @pl.when(active)
                def _():
                    psid = jnp.where(owner, sid + 8, sid - 8)
                    pltpu.sync_copy(sh.at[pl.ds(psid, 1)], t_v)

                    @plsc.parallel_loop(0, N // _LANES, unroll=8)
                    def _add(i):
                        sl = pl.ds(i * _LANES, _LANES)
                        dst_v[0, sl] = dst_v[0, sl] + t_v[0, sl]
                plsc.subcore_barrier()

            @pl.when(active)
            def _():
                pltpu.sync_copy(x_hbm.at[pl.ds(f, 1)], a_v)
                zero(b_v)
                edge_pass(a_v, b_v, ch0, half)
            merge(b_v)

            @pl.when(owner)
            def _():
                pltpu.sync_copy(b_v, y1_hbm.at[pl.ds(f, 1)])

            @pl.when(active)
            def _():
                zero(a_v)
                edge_pass(b_v, a_v, ch0, half)
            merge(a_v)

            @pl.when(owner)
            def _():
                pltpu.sync_copy(a_v, y2_hbm.at[pl.ds(f, 1)])
    elif mode == "pair":
        @functools.partial(
            pl.kernel,
            out_type=(jax.ShapeDtypeStruct((F, N), jnp.float32),
                      jax.ShapeDtypeStruct((F, N), jnp.float32)),
            mesh=mesh,
            compiler_params=pltpu.CompilerParams(needs_layout_passes=False),
            scratch_types=[
                pltpu.VMEM((fpw, N), jnp.float32),
                pltpu.VMEM((fpw, N), jnp.float32),
                pltpu.VMEM((_NBUF, chunk), jnp.int32),
                pltpu.VMEM((_NBUF, chunk), jnp.float32),
                pltpu.SemaphoreType.DMA,
                pltpu.SemaphoreType.DMA,
            ],
        )
        def spmm(x_hbm, rc_hbm, vals_hbm, y1_hbm, y2_hbm,
                 a_v, b_v, rc_v, v_v, sem0, sem1):
            wid = lax.axis_index("s") * _NC + lax.axis_index("c")
            zero, edge_pass = make_helpers(rc_hbm, vals_hbm, rc_v, v_v,
                                           (sem0, sem1))

            @pl.when(wid < nwact)
            def _():
                f0 = wid * fpw
                pltpu.sync_copy(x_hbm.at[pl.ds(f0, fpw)], a_v)
                zero(b_v)
                edge_pass(a_v, b_v)
                pltpu.sync_copy(b_v, y1_hbm.at[pl.ds(f0, fpw)])
                zero(a_v)
                edge_pass(b_v, a_v)
                pltpu.sync_copy(a_v, y2_hbm.at[pl.ds(f0, fpw)])
    else:
        @functools.partial(
            pl.kernel,
            out_type=jax.ShapeDtypeStruct((F, N), jnp.float32),
            mesh=mesh,
            compiler_params=pltpu.CompilerParams(needs_layout_passes=False),
            scratch_types=[
                pltpu.VMEM((fpw, N), jnp.float32),
                pltpu.VMEM((fpw, N), jnp.float32),
                pltpu.VMEM((_NBUF, chunk), jnp.int32),
                pltpu.VMEM((_NBUF, chunk), jnp.float32),
                pltpu.SemaphoreType.DMA,
                pltpu.SemaphoreType.DMA,
            ],
        )
        def spmm(u1_hbm, u2_hbm, rc_hbm, vals_hbm, r_hbm,
                 a_v, b_v, rc_v, v_v, sem0, sem1):
            wid = lax.axis_index("s") * _NC + lax.axis_index("c")
            zero, edge_pass = make_helpers(rc_hbm, vals_hbm, rc_v, v_v,
                                           (sem0, sem1))

            @pl.when(wid < nwact)
            def _():
                f0 = wid * fpw
                pltpu.sync_copy(u2_hbm.at[pl.ds(f0, fpw)], a_v)
                zero(b_v)
                edge_pass(a_v, b_v)
                pltpu.sync_copy(u1_hbm.at[pl.ds(f0, fpw)], a_v)

                @plsc.parallel_loop(0, N // _LANES, unroll=8)
                def _acc(i):
                    sl = pl.ds(i * _LANES, _LANES)
                    for j in range(fpw):
                        b_v[j, sl] = a_v[j, sl] + 2.0 * b_v[j, sl]

                zero(a_v)
                edge_pass(b_v, a_v)
                pltpu.sync_copy(a_v, r_hbm.at[pl.ds(f0, fpw)])

    return spmm


def _bn_from_relu(nb, n_nodes, h, g, beta):
    fo = h.shape[0]
    och = fo // nb
    s1 = jnp.sum(h, axis=1, keepdims=True)
    s2 = jnp.sum(h * h, axis=1, keepdims=True)
    s1o = s1[0:och]
    s2o = s2[0:och]
    for b in range(1, nb):
        s1o = s1o + s1[b * och:(b + 1) * och]
        s2o = s2o + s2[b * och:(b + 1) * och]
    cnt = nb * n_nodes
    m = s1o / cnt
    var = s2o / cnt - m * m
    inv = lax.rsqrt(var + 1e-5)
    scale_o = inv * g
    shift_o = beta - m * scale_o
    scale = jnp.concatenate([scale_o] * nb, axis=0)
    shift = jnp.concatenate([shift_o] * nb, axis=0)
    return h * scale + shift


def _combine_body(nb, n_nodes, x_ref, y1_ref, y2_ref, w0_ref, w1_ref,
                  w2_ref, bias_ref, g_ref, beta_ref, o_ref):
    h = jnp.dot(w0_ref[...], x_ref[...], preferred_element_type=jnp.float32)
    h = h + jnp.dot(w1_ref[...], y1_ref[...], preferred_element_type=jnp.float32)
    h = h + jnp.dot(w2_ref[...], y2_ref[...], preferred_element_type=jnp.float32)
    h = jnp.maximum(h + bias_ref[...], 0.0)
    o_ref[...] = _bn_from_relu(nb, n_nodes, h, g_ref[...], beta_ref[...])


def _combine_u_body(nb, n_nodes, x_ref, y1_ref, y2_ref, w0_ref, w1_ref,
                    w2_ref, bias_ref, g_ref, beta_ref, wu1_ref, wu2_ref,
                    o_ref, u1_ref, u2_ref):
    h = jnp.dot(w0_ref[...], x_ref[...], preferred_element_type=jnp.float32)
    h = h + jnp.dot(w1_ref[...], y1_ref[...], preferred_element_type=jnp.float32)
    h = h + jnp.dot(w2_ref[...], y2_ref[...], preferred_element_type=jnp.float32)
    h = jnp.maximum(h + bias_ref[...], 0.0)
    out = _bn_from_relu(nb, n_nodes, h, g_ref[...], beta_ref[...])
    o_ref[...] = out
    u1_ref[...] = jnp.dot(wu1_ref[...], out, preferred_element_type=jnp.float32)
    u2_ref[...] = jnp.dot(wu2_ref[...], out, preferred_element_type=jnp.float32)


def _final_body(nb, x_ref, r_ref, w0_ref, bias_ref, o_ref):
    h = jnp.dot(w0_ref[...], x_ref[...], preferred_element_type=jnp.float32)
    h = h + r_ref[...]
    h = jnp.maximum(h + bias_ref[...], 0.0)
    p = jnp.max(h, axis=1, keepdims=True)
    fo = p.shape[0]
    och = fo // nb
    segs = []
    for b in range(nb):
        seg = p[b * och:(b + 1) * och]
        mx = jnp.max(seg)
        lse = jnp.log(jnp.sum(jnp.exp(seg - mx))) + mx
        segs.append(seg - lse)
    o_ref[...] = jnp.concatenate(segs, axis=0)


def _block_diag_weights(A, nb):
    c, o = A.shape
    eye = jnp.eye(nb, dtype=A.dtype)
    return jnp.einsum("bd,co->bodc", eye, A).reshape(nb * o, nb * c)


def _cheb_weights(W, nb):
    a0 = W[0] - W[2]
    a1 = W[1]
    a2 = 2.0 * W[2]
    return (_block_diag_weights(a0, nb), _block_diag_weights(a1, nb),
            _block_diag_weights(a2, nb))


def kernel(x, lap_rows, lap_cols, lap_vals, W_in, b_in, bn1_g, bn1_b,
           W_h, b_h, bn2_g, bn2_b, W_out, b_out):
    nb, c_in, n = x.shape
    e2 = lap_rows.shape[0]
    step = _CHUNK * _NBUF
    e2_eff = -(-e2 // step) * step
    e2p = -(-e2 // (2 * step)) * (2 * step)
    pad = e2p - e2
    rc = jnp.bitwise_or(jnp.left_shift(lap_rows.astype(jnp.int32), 14),
                        lap_cols.astype(jnp.int32))
    rc_p = jnp.concatenate([rc, jnp.zeros((pad,), jnp.int32)])
    vals_p = jnp.concatenate([lap_vals, jnp.zeros((pad,), lap_vals.dtype)])

    def spmm_pair(xf):
        f = xf.shape[0]
        if 2 * f <= _NW and f % _NC == 0:
            return _make_spmm(f, n, e2p, "pair_split")(xf, rc_p, vals_p)
        return _make_spmm(f, n, e2p, "pair", 2 * _CHUNK)(xf, rc_p, vals_p)

    def layer1(xf, W, bias, g, beta):
        och = W.shape[2]
        fo = nb * och
        y1, y2 = spmm_pair(xf)
        w0b, w1b, w2b = _cheb_weights(W, nb)
        bias_v = jnp.tile(bias, nb).reshape(fo, 1)
        return pl.pallas_call(
            functools.partial(_combine_body, nb, n),
            out_shape=jax.ShapeDtypeStruct((fo, n), jnp.float32),
        )(xf, y1, y2, w0b, w1b, w2b, bias_v,
          g.reshape(och, 1), beta.reshape(och, 1))

    def layer2_with_proj(xf, W, bias, g, beta, W_next):
        och = W.shape[2]
        fo = nb * och
        fu = nb * W_next.shape[2]
        y1, y2 = spmm_pair(xf)
        w0b, w1b, w2b = _cheb_weights(W, nb)
        wu1 = _block_diag_weights(W_next[1], nb)
        wu2 = _block_diag_weights(W_next[2], nb)
        bias_v = jnp.tile(bias, nb).reshape(fo, 1)
        return pl.pallas_call(
            functools.partial(_combine_u_body, nb, n),
            out_shape=(jax.ShapeDtypeStruct((fo, n), jnp.float32),
                       jax.ShapeDtypeStruct((fu, n), jnp.float32),
                       jax.ShapeDtypeStruct((fu, n), jnp.float32)),
        )(xf, y1, y2, w0b, w1b, w2b, bias_v,
          g.reshape(och, 1), beta.reshape(och, 1), wu1, wu2)

    def final_layer(xf, u1, u2, W, bias):
        och = W.shape[2]
        fo = nb * och
        r = _make_spmm(fo, n, e2p, "l3", 2 * _CHUNK)(u1, u2, rc_p, vals_p)
        w0b = _block_diag_weights(W[0] - W[2], nb)
        bias_v = jnp.tile(bias, nb).reshape(fo, 1)
        p = pl.pallas_call(
            functools.partial(_final_body, nb),
            out_shape=jax.ShapeDtypeStruct((fo, 1), jnp.float32),
        )(xf, r, w0b, bias_v)
        return p.reshape(nb, och)

    x0 = x.reshape(nb * c_in, n)
    h1 = layer1(x0, W_in, b_in, bn1_g, bn1_b)
    h2, u1, u2 = layer2_with_proj(h1, W_h, b_h, bn2_g, bn2_b, W_out)
    return final_layer(h2, u1, u2, W_out, b_out)


# speedup vs baseline: 1.2681x; 1.2681x over previous
import functools

import jax
import jax.numpy as jnp
from jax import lax
from jax.experimental import pallas as pl
from jax.experimental.pallas import tpu as pltpu
from jax.experimental.pallas import tpu_sc as plsc

_NC = 2
_NS = 16
_NW = _NC * _NS
_LANES = 16
_CHUNK = 2048
_NBUF = 2


@functools.lru_cache(maxsize=None)
def _make_spmm(F, N, E2eff, mode="pair", chunk=_CHUNK):
    fpw = -(-F // _NW)
    nwact = -(-F // fpw)
    nchunks = E2eff // chunk
    groups = chunk // _LANES

    mesh = plsc.VectorSubcoreMesh(
        core_axis_name="c", subcore_axis_name="s",
        num_cores=_NC, num_subcores=_NS)

    def make_helpers(rc_hbm, vals_hbm, rc_v, v_v, sems):
        def start(ch, b):
            off = ch * chunk
            pltpu.async_copy(rc_hbm.at[pl.ds(off, chunk)], rc_v.at[b], sems[b])
            pltpu.async_copy(vals_hbm.at[pl.ds(off, chunk)], v_v.at[b], sems[b])

        def drain(b):
            pltpu.make_async_copy(rc_hbm.at[pl.ds(0, chunk)], rc_v.at[b], sems[b]).wait()
            pltpu.make_async_copy(vals_hbm.at[pl.ds(0, chunk)], v_v.at[b], sems[b]).wait()

        def zero(dst_v):
            z = jnp.zeros((_LANES,), jnp.float32)

            @plsc.parallel_loop(0, N // _LANES, unroll=8)
            def zbody(i):
                for j in range(fpw):
                    dst_v[j, pl.ds(i * _LANES, _LANES)] = z

        def edge_pass(src_v, dst_v, ch_base=0, nch=nchunks):
            start(ch_base, 0)

            def compute(b):
                @plsc.parallel_loop(0, groups, unroll=8)
                def body(gi):
                    base = gi * _LANES
                    rc = rc_v[b, pl.ds(base, _LANES)]
                    rr = lax.shift_right_logical(rc, 14)
                    cc = lax.bitwise_and(rc, 16383)
                    vv = v_v[b, pl.ds(base, _LANES)]
                    gs = []
                    for j in range(fpw):
                        jf = jnp.full((_LANES,), j, jnp.int32)
                        gs.append(plsc.load_gather(src_v, [jf, cc]) * vv)
                    for j in range(fpw):
                        jf = jnp.full((_LANES,), j, jnp.int32)
                        plsc.addupdate_scatter(dst_v, [jf, rr], gs[j])

            def outer(g, carry):
                for b in range(_NBUF):
                    ch = g * _NBUF + b

                    @pl.when(ch + 1 < nch)
                    def _():
                        start(ch_base + ch + 1, 1 - b)

                    drain(b)
                    compute(b)
                return carry
            lax.fori_loop(0, nch // _NBUF, outer, 0)

        return zero, edge_pass

    if mode == "pair_split":
        fsc = F // _NC
        half = nchunks // 2

        @functools.partial(
            pl.kernel,
            out_type=(jax.ShapeDtypeStruct((F, N), jnp.float32),
                      jax.ShapeDtypeStruct((F, N), jnp.float32)),
            mesh=mesh,
            compiler_params=pltpu.CompilerParams(needs_layout_passes=False),
            scratch_types=[
                pltpu.VMEM((1, N), jnp.float32),
                pltpu.VMEM((1, N), jnp.float32),
                pltpu.VMEM((1, N), jnp.float32),
                pltpu.VMEM_SHARED((_NS, N), jnp.float32),
                pltpu.VMEM((_NBUF, chunk), jnp.int32),
                pltpu.VMEM((_NBUF, chunk), jnp.float32),
                pltpu.SemaphoreType.DMA,
                pltpu.SemaphoreType.DMA,
            ],
        )
        def spmm(x_hbm, rc_hbm, vals_hbm, y1_hbm, y2_hbm,
                 a_v, b_v, t_v, sh, rc_v, v_v, sem0, sem1):
            sid = lax.axis_index("s")
            cid = lax.axis_index("c")
            zero, edge_pass = make_helpers(rc_hbm, vals_hbm, rc_v, v_v,
                                           (sem0, sem1))
            owner = sid < fsc
            helper = jnp.logical_and(sid >= 8, sid < 8 + fsc)
            active = jnp.logical_or(owner, helper)
            floc = jnp.where(owner, sid, sid - 8)
            f = cid * fsc + floc
            ch0 = jnp.where(owner, 0, half)

            def merge(dst_v):
                @pl.when(active)
                def _():
                    pltpu.sync_copy(dst_v, sh.at[pl.ds(sid, 1)])
                plsc.subcore_barrier()

                @pl.when(active)
                def _():
                    psid = jnp.where(owner, sid + 8, sid - 8)
                    pltpu.sync_copy(sh.at[pl.ds(psid, 1)], t_v)

                    @plsc.parallel_loop(0, N // _LANES, unroll=8)
                    def _add(i):
                        sl = pl.ds(i * _LANES, _LANES)
                        dst_v[0, sl] = dst_v[0, sl] + t_v[0, sl]
                plsc.subcore_barrier()

            @pl.when(active)
            def _():
                pltpu.sync_copy(x_hbm.at[pl.ds(f, 1)], a_v)
                zero(b_v)
                edge_pass(a_v, b_v, ch0, half)
            merge(b_v)

            @pl.when(owner)
            def _():
                pltpu.sync_copy(b_v, y1_hbm.at[pl.ds(f, 1)])

            @pl.when(active)
            def _():
                zero(a_v)
                edge_pass(b_v, a_v, ch0, half)
            merge(a_v)

            @pl.when(owner)
            def _():
                pltpu.sync_copy(a_v, y2_hbm.at[pl.ds(f, 1)])
    elif mode == "pair":
        @functools.partial(
            pl.kernel,
            out_type=(jax.ShapeDtypeStruct((F, N), jnp.float32),
                      jax.ShapeDtypeStruct((F, N), jnp.float32)),
            mesh=mesh,
            compiler_params=pltpu.CompilerParams(needs_layout_passes=False),
            scratch_types=[
                pltpu.VMEM((fpw, N), jnp.float32),
                pltpu.VMEM((fpw, N), jnp.float32),
                pltpu.VMEM((_NBUF, chunk), jnp.int32),
                pltpu.VMEM((_NBUF, chunk), jnp.float32),
                pltpu.SemaphoreType.DMA,
                pltpu.SemaphoreType.DMA,
            ],
        )
        def spmm(x_hbm, rc_hbm, vals_hbm, y1_hbm, y2_hbm,
                 a_v, b_v, rc_v, v_v, sem0, sem1):
            wid = lax.axis_index("s") * _NC + lax.axis_index("c")
            zero, edge_pass = make_helpers(rc_hbm, vals_hbm, rc_v, v_v,
                                           (sem0, sem1))

            @pl.when(wid < nwact)
            def _():
                f0 = wid * fpw
                pltpu.sync_copy(x_hbm.at[pl.ds(f0, fpw)], a_v)
                zero(b_v)
                edge_pass(a_v, b_v)
                pltpu.sync_copy(b_v, y1_hbm.at[pl.ds(f0, fpw)])
                zero(a_v)
                edge_pass(b_v, a_v)
                pltpu.sync_copy(a_v, y2_hbm.at[pl.ds(f0, fpw)])
    else:
        @functools.partial(
            pl.kernel,
            out_type=jax.ShapeDtypeStruct((F, N), jnp.float32),
            mesh=mesh,
            compiler_params=pltpu.CompilerParams(needs_layout_passes=False),
            scratch_types=[
                pltpu.VMEM((fpw, N), jnp.float32),
                pltpu.VMEM((fpw, N), jnp.float32),
                pltpu.VMEM((_NBUF, chunk), jnp.int32),
                pltpu.VMEM((_NBUF, chunk), jnp.float32),
                pltpu.SemaphoreType.DMA,
                pltpu.SemaphoreType.DMA,
            ],
        )
        def spmm(u1_hbm, u2_hbm, rc_hbm, vals_hbm, r_hbm,
                 a_v, b_v, rc_v, v_v, sem0, sem1):
            wid = lax.axis_index("s") * _NC + lax.axis_index("c")
            zero, edge_pass = make_helpers(rc_hbm, vals_hbm, rc_v, v_v,
                                           (sem0, sem1))

            @pl.when(wid < nwact)
            def _():
                f0 = wid * fpw
                pltpu.sync_copy(u2_hbm.at[pl.ds(f0, fpw)], a_v)
                zero(b_v)
                edge_pass(a_v, b_v)
                pltpu.sync_copy(u1_hbm.at[pl.ds(f0, fpw)], a_v)

                @plsc.parallel_loop(0, N // _LANES, unroll=8)
                def _acc(i):
                    sl = pl.ds(i * _LANES, _LANES)
                    for j in range(fpw):
                        b_v[j, sl] = a_v[j, sl] + 2.0 * b_v[j, sl]

                zero(a_v)
                edge_pass(b_v, a_v)
                pltpu.sync_copy(a_v, r_hbm.at[pl.ds(f0, fpw)])

    return spmm


def _bn_from_relu(nb, n_nodes, h, g, beta):
    fo = h.shape[0]
    och = fo // nb
    s1 = jnp.sum(h, axis=1, keepdims=True)
    s2 = jnp.sum(h * h, axis=1, keepdims=True)
    s1o = s1[0:och]
    s2o = s2[0:och]
    for b in range(1, nb):
        s1o = s1o + s1[b * och:(b + 1) * och]
        s2o = s2o + s2[b * och:(b + 1) * och]
    cnt = nb * n_nodes
    m = s1o / cnt
    var = s2o / cnt - m * m
    inv = lax.rsqrt(var + 1e-5)
    scale_o = inv * g
    shift_o = beta - m * scale_o
    scale = jnp.concatenate([scale_o] * nb, axis=0)
    shift = jnp.concatenate([shift_o] * nb, axis=0)
    return h * scale + shift


def _combine_body(nb, n_nodes, x_ref, y1_ref, y2_ref, w0_ref, w1_ref,
                  w2_ref, bias_ref, g_ref, beta_ref, o_ref):
    h = jnp.dot(w0_ref[...], x_ref[...], preferred_element_type=jnp.float32)
    h = h + jnp.dot(w1_ref[...], y1_ref[...], preferred_element_type=jnp.float32)
    h = h + jnp.dot(w2_ref[...], y2_ref[...], preferred_element_type=jnp.float32)
    h = jnp.maximum(h + bias_ref[...], 0.0)
    o_ref[...] = _bn_from_relu(nb, n_nodes, h, g_ref[...], beta_ref[...])


def _combine_u_body(nb, n_nodes, x_ref, y1_ref, y2_ref, w0_ref, w1_ref,
                    w2_ref, bias_ref, g_ref, beta_ref, wu1_ref, wu2_ref,
                    o_ref, u1_ref, u2_ref):
    h = jnp.dot(w0_ref[...], x_ref[...], preferred_element_type=jnp.float32)
    h = h + jnp.dot(w1_ref[...], y1_ref[...], preferred_element_type=jnp.float32)
    h = h + jnp.dot(w2_ref[...], y2_ref[...], preferred_element_type=jnp.float32)
    h = jnp.maximum(h + bias_ref[...], 0.0)
    out = _bn_from_relu(nb, n_nodes, h, g_ref[...], beta_ref[...])
    o_ref[...] = out
    u1_ref[...] = jnp.dot(wu1_ref[...], out, preferred_element_type=jnp.float32)
    u2_ref[...] = jnp.dot(wu2_ref[...], out, preferred_element_type=jnp.float32)


def _final_body(nb, x_ref, r_ref, w0_ref, bias_ref, o_ref):
    h = jnp.dot(w0_ref[...], x_ref[...], preferred_element_type=jnp.float32)
    h = h + r_ref[...]
    h = jnp.maximum(h + bias_ref[...], 0.0)
    p = jnp.max(h, axis=1, keepdims=True)
    fo = p.shape[0]
    och = fo // nb
    segs = []
    for b in range(nb):
        seg = p[b * och:(b + 1) * och]
        mx = jnp.max(seg)
        lse = jnp.log(jnp.sum(jnp.exp(seg - mx))) + mx
        segs.append(seg - lse)
    o_ref[...] = jnp.concatenate(segs, axis=0)


def _block_diag_weights(A, nb):
    c, o = A.shape
    eye = jnp.eye(nb, dtype=A.dtype)
    return jnp.einsum("bd,co->bodc", eye, A).reshape(nb * o, nb * c)


def _cheb_weights(W, nb):
    a0 = W[0] - W[2]
    a1 = W[1]
    a2 = 2.0 * W[2]
    return (_block_diag_weights(a0, nb), _block_diag_weights(a1, nb),
            _block_diag_weights(a2, nb))


def kernel(x, lap_rows, lap_cols, lap_vals, W_in, b_in, bn1_g, bn1_b,
           W_h, b_h, bn2_g, bn2_b, W_out, b_out):
    nb, c_in, n = x.shape
    e2 = lap_rows.shape[0]
    step = _CHUNK * _NBUF
    e2_eff = -(-e2 // step) * step
    e2p = -(-e2 // (2 * step)) * (2 * step)
    pad = e2p - e2
    rc = jnp.bitwise_or(jnp.left_shift(lap_rows.astype(jnp.int32), 14),
                        lap_cols.astype(jnp.int32))
    rc_p = jnp.concatenate([rc, jnp.zeros((pad,), jnp.int32)])
    vals_p = jnp.concatenate([lap_vals, jnp.zeros((pad,), lap_vals.dtype)])

    def spmm_pair(xf):
        f = xf.shape[0]
        if 2 * f <= _NW and f % _NC == 0:
            return _make_spmm(f, n, e2p, "pair_split")(xf, rc_p, vals_p)
        return _make_spmm(f, n, e2_eff, "pair")(xf, rc_p, vals_p)

    def layer1(xf, W, bias, g, beta):
        och = W.shape[2]
        fo = nb * och
        y1, y2 = spmm_pair(xf)
        w0b, w1b, w2b = _cheb_weights(W, nb)
        bias_v = jnp.tile(bias, nb).reshape(fo, 1)
        return pl.pallas_call(
            functools.partial(_combine_body, nb, n),
            out_shape=jax.ShapeDtypeStruct((fo, n), jnp.float32),
        )(xf, y1, y2, w0b, w1b, w2b, bias_v,
          g.reshape(och, 1), beta.reshape(och, 1))

    def layer2_with_proj(xf, W, bias, g, beta, W_next):
        och = W.shape[2]
        fo = nb * och
        fu = nb * W_next.shape[2]
        y1, y2 = spmm_pair(xf)
        w0b, w1b, w2b = _cheb_weights(W, nb)
        wu1 = _block_diag_weights(W_next[1], nb)
        wu2 = _block_diag_weights(W_next[2], nb)
        bias_v = jnp.tile(bias, nb).reshape(fo, 1)
        return pl.pallas_call(
            functools.partial(_combine_u_body, nb, n),
            out_shape=(jax.ShapeDtypeStruct((fo, n), jnp.float32),
                       jax.ShapeDtypeStruct((fu, n), jnp.float32),
                       jax.ShapeDtypeStruct((fu, n), jnp.float32)),
        )(xf, y1, y2, w0b, w1b, w2b, bias_v,
          g.reshape(och, 1), beta.reshape(och, 1), wu1, wu2)

    def final_layer(xf, u1, u2, W, bias):
        och = W.shape[2]
        fo = nb * och
        r = _make_spmm(fo, n, e2_eff, "l3")(u1, u2, rc_p, vals_p)
        w0b = _block_diag_weights(W[0] - W[2], nb)
        bias_v = jnp.tile(bias, nb).reshape(fo, 1)
        p = pl.pallas_call(
            functools.partial(_final_body, nb),
            out_shape=jax.ShapeDtypeStruct((fo, 1), jnp.float32),
        )(xf, r, w0b, bias_v)
        return p.reshape(nb, och)

    x0 = x.reshape(nb * c_in, n)
    h1 = layer1(x0, W_in, b_in, bn1_g, bn1_b)
    h2, u1, u2 = layer2_with_proj(h1, W_h, b_h, bn2_g, bn2_b, W_out)
    return final_layer(h2, u1, u2, W_out, b_out)
